# baseline (speedup 1.0000x reference)
import functools

import jax
import jax.numpy as jnp
from jax import lax
from jax.experimental import pallas as pl
from jax.experimental.pallas import tpu as pltpu
from jax.experimental.pallas import tpu_sc as plsc

_NUM_CORES = 2
_NUM_SUBCORES = 16
_NB = 256
_L = 16


@functools.cache
def _make_gather(N, S, D):
    nw = _NUM_CORES * _NUM_SUBCORES
    n_items = S * (N // _NB)
    assert n_items % (2 * nw) == 0
    per_w = n_items // nw
    blocks_per_s = N // _NB
    mesh = plsc.VectorSubcoreMesh(core_axis_name="c", subcore_axis_name="s")

    @functools.partial(
        pl.kernel,
        out_type=jax.ShapeDtypeStruct((S, D, N), jnp.float32),
        mesh=mesh,
        scratch_types=[
            pltpu.VMEM((_NB,), jnp.int32),
            pltpu.VMEM((_NB,), jnp.int32),
            pltpu.VMEM((_NB,), jnp.int32),
            pltpu.VMEM((2, _NB), jnp.int32),
            pltpu.VMEM((2, _NB, 128), jnp.float32),
            pltpu.VMEM((D, _NB), jnp.float32),
            pltpu.SemaphoreType.DMA((2,)),
        ],
        compiler_params=pltpu.CompilerParams(needs_layout_passes=False),
    )
    def gather_kernel(tok_hbm, table_hbm, out_hbm, tok_v, rid0_v, rid1_v,
                      half_v, rows_v, out_v, gsem):
        rid_refs = (rid0_v, rid1_v)
        wid = lax.axis_index("s") * _NUM_CORES + lax.axis_index("c")
        lane = lax.broadcasted_iota(jnp.int32, (_L,), 0)

        def stage_and_start(m, b):
            s = m // blocks_per_s
            nb0 = (m % blocks_per_s) * _NB
            pltpu.sync_copy(tok_hbm.at[pl.ds(s * N + nb0, _NB)], tok_v)
            for g in range(_NB // _L):
                tv = tok_v[pl.ds(g * _L, _L)]
                rid_refs[b][pl.ds(g * _L, _L)] = tv >> 1
                half_v[b, pl.ds(g * _L, _L)] = (tv & 1) * 64
            pltpu.async_copy(table_hbm.at[rid_refs[b]], rows_v.at[b],
                             gsem.at[b])

        def finish(m, b):
            pltpu.make_async_copy(table_hbm.at[rid_refs[b]], rows_v.at[b],
                                  gsem.at[b]).wait()

            def compact(e, carry):
                for g in range(_NB // _L):
                    kv = lane + (g * _L)
                    cv = half_v[b, pl.ds(g * _L, _L)] + e
                    out_v[e, pl.ds(g * _L, _L)] = plsc.load_gather(
                        rows_v.at[b], [kv, cv])
                return carry

            lax.fori_loop(0, D, compact, 0)
            s = m // blocks_per_s
            nb0 = (m % blocks_per_s) * _NB
            pltpu.sync_copy(out_v, out_hbm.at[s, :, pl.ds(nb0, _NB)])

        base = wid * per_w
        stage_and_start(base, 0)

        def body(j, carry):
            for b in (0, 1):
                l = 2 * j + b
                if b == 0:
                    stage_and_start(base + l + 1, 1)
                else:

                    @pl.when(j < per_w // 2 - 1)
                    def _():
                        stage_and_start(base + l + 1, 0)

                finish(base + l, b)
            return carry

        lax.fori_loop(0, per_w // 2, body, 0)

    return gather_kernel


def kernel(token_ids, weight):
    bsz, seq = token_ids.shape
    _, d = weight.shape
    toks = token_ids.T.reshape(-1)
    table2 = weight.reshape(weight.shape[0] // 2, 128)
    out_phys = _make_gather(bsz, seq, d)(toks, table2)
    return jnp.transpose(out_phys, (2, 0, 1))


# speedup vs baseline: 1.3480x; 1.3480x over previous
import functools

import jax
import jax.numpy as jnp
from jax import lax
from jax.experimental import pallas as pl
from jax.experimental.pallas import tpu as pltpu
from jax.experimental.pallas import tpu_sc as plsc

_NUM_CORES = 2
_NUM_SUBCORES = 16
_NB = 256
_L = 16


@functools.cache
def _make_gather(N, S, D):
    nw = _NUM_CORES * _NUM_SUBCORES
    n_items = S * (N // _NB)
    assert n_items % (2 * nw) == 0
    per_w = n_items // nw
    blocks_per_s = N // _NB
    mesh = plsc.VectorSubcoreMesh(core_axis_name="c", subcore_axis_name="s")

    @functools.partial(
        pl.kernel,
        out_type=jax.ShapeDtypeStruct((S, D, N), jnp.float32),
        mesh=mesh,
        scratch_types=[
            pltpu.VMEM((_NB,), jnp.int32),
            pltpu.VMEM((_NB,), jnp.int32),
            pltpu.VMEM((_NB,), jnp.int32),
            pltpu.VMEM((2, _NB), jnp.int32),
            pltpu.VMEM((2, _NB, 128), jnp.float32),
            pltpu.VMEM((D, _NB), jnp.float32),
            pltpu.SemaphoreType.DMA((2,)),
        ],
        compiler_params=pltpu.CompilerParams(needs_layout_passes=False),
    )
    def gather_kernel(tok_hbm, table_hbm, out_hbm, tok_v, rid0_v, rid1_v,
                      half_v, rows_v, out_v, gsem):
        rid_refs = (rid0_v, rid1_v)
        wid = lax.axis_index("s") * _NUM_CORES + lax.axis_index("c")
        lane = lax.broadcasted_iota(jnp.int32, (_L,), 0)

        def stage_and_start(m, b):
            s = m // blocks_per_s
            nb0 = (m % blocks_per_s) * _NB
            pltpu.sync_copy(tok_hbm.at[pl.ds(s * N + nb0, _NB)], tok_v)
            for g in range(_NB // _L):
                tv = tok_v[pl.ds(g * _L, _L)]
                rid_refs[b][pl.ds(g * _L, _L)] = tv >> 1
                half_v[b, pl.ds(g * _L, _L)] = (tv & 1) * 64
            pltpu.async_copy(table_hbm.at[rid_refs[b]], rows_v.at[b],
                             gsem.at[b])

        def finish(m, b):
            pltpu.make_async_copy(table_hbm.at[rid_refs[b]], rows_v.at[b],
                                  gsem.at[b]).wait()

            def compact(g, carry):
                kv = lane + g * _L
                cv0 = half_v[b, pl.ds(g * _L, _L)]
                for e in range(D):
                    out_v[e, pl.ds(g * _L, _L)] = plsc.load_gather(
                        rows_v.at[b], [kv, cv0 + e])
                return carry

            lax.fori_loop(0, _NB // _L, compact, 0)
            s = m // blocks_per_s
            nb0 = (m % blocks_per_s) * _NB
            pltpu.sync_copy(out_v, out_hbm.at[s, :, pl.ds(nb0, _NB)])

        base = wid * per_w
        stage_and_start(base, 0)

        def body(j, carry):
            for b in (0, 1):
                l = 2 * j + b
                if b == 0:
                    stage_and_start(base + l + 1, 1)
                else:

                    @pl.when(j < per_w // 2 - 1)
                    def _():
                        stage_and_start(base + l + 1, 0)

                finish(base + l, b)
            return carry

        lax.fori_loop(0, per_w // 2, body, 0)

    return gather_kernel


def kernel(token_ids, weight):
    bsz, seq = token_ids.shape
    _, d = weight.shape
    toks = token_ids.T.reshape(-1)
    table2 = weight.reshape(weight.shape[0] // 2, 128)
    out_phys = _make_gather(bsz, seq, d)(toks, table2)
    return jnp.transpose(out_phys, (2, 0, 1))


# speedup vs baseline: 2.1445x; 1.5909x over previous
import functools

import jax
import jax.numpy as jnp
from jax import lax
from jax.experimental import pallas as pl
from jax.experimental.pallas import tpu as pltpu
from jax.experimental.pallas import tpu_sc as plsc

_NUM_CORES = 2
_NUM_SUBCORES = 16
_CHUNK = 512


@functools.cache
def _make_gather(B, D):
    nw = _NUM_CORES * _NUM_SUBCORES
    assert B % (8 * nw) == 0
    b_per_w = B // nw
    assert b_per_w % (2 * _CHUNK) == 0
    n_loops = b_per_w // (2 * _CHUNK)
    mesh = plsc.VectorSubcoreMesh(core_axis_name="c", subcore_axis_name="s")

    @functools.partial(
        pl.kernel,
        out_type=jax.ShapeDtypeStruct((B, D), jnp.float32),
        mesh=mesh,
        scratch_types=[
            pltpu.VMEM((b_per_w,), jnp.int32),
            pltpu.VMEM((2, _CHUNK, D), jnp.float32),
            pltpu.SemaphoreType.DMA((2,)),
        ],
        compiler_params=pltpu.CompilerParams(use_tc_tiling_on_sc=False),
    )
    def gather_kernel(idx_hbm, table_hbm, out_hbm, idx_v, rows_v, gsem):
        wid = lax.axis_index("s") * _NUM_CORES + lax.axis_index("c")
        base = pl.multiple_of(wid * b_per_w, 8)

        pltpu.sync_copy(idx_hbm.at[pl.ds(base, b_per_w)], idx_v)

        def start_gather(c, b):
            loc = pl.multiple_of(c * _CHUNK, 8)
            pltpu.async_copy(
                table_hbm.at[idx_v.at[pl.ds(loc, _CHUNK)]], rows_v.at[b],
                gsem.at[b])

        def wait_gather(c, b):
            loc = pl.multiple_of(c * _CHUNK, 8)
            pltpu.make_async_copy(
                table_hbm.at[idx_v.at[pl.ds(loc, _CHUNK)]], rows_v.at[b],
                gsem.at[b]).wait()

        start_gather(0, 0)

        def body(j, carry):
            for b in (0, 1):
                c = 2 * j + b
                wait_gather(c, b)
                if b == 0:
                    start_gather(c + 1, 1)
                else:

                    @pl.when(j < n_loops - 1)
                    def _():
                        start_gather(c + 1, 0)

                glob = pl.multiple_of(base + c * _CHUNK, 8)
                pltpu.sync_copy(rows_v.at[b], out_hbm.at[pl.ds(glob, _CHUNK)])
            return carry

        lax.fori_loop(0, n_loops, body, 0)

    return gather_kernel


def kernel(token_ids, weight):
    bsz, seq = token_ids.shape
    _, d = weight.shape
    flat = token_ids.reshape(bsz * seq).astype(jnp.int32)
    out = _make_gather(bsz * seq, d)(flat, weight)
    return out.reshape(bsz, seq, d)
